# baseline (speedup 1.0000x reference)
import functools

import jax
import jax.numpy as jnp
from jax import lax
from jax.experimental import pallas as pl
from jax.experimental.pallas import tpu as pltpu
from jax.experimental.pallas import tpu_sc as plsc

N = 10000
E = 320000
F = 128
FH = 64
NB = 16

NC = 2
NS = 16
NW = NC * NS

NPT = 320
TRASH = NPT
ACC_ROWS = NPT + 1
KE = 4000
NBLK = E // KE
CH = 128

NEG = -3.0e38
POS = 3.0e38

f32 = jnp.float32
i32 = jnp.int32


CPB = (KE + CH - 1) // CH
EPAD = NBLK * CPB * CH
SB = 16
SBCH = SB * CH
D = 3
BR = 2


def _accum_chunk(rows, slot, dbuf, doff, sumacc, maxacc, minacc, cntacc,
                 with_cnt, onev):
    def _egrp(g, _):
        dl16 = dbuf[pl.ds(doff + g * 16, 16)]
        for lane in range(16):
            dl = dl16[lane]
            e = g * 16 + lane
            for r in range(FH // 16):
                rv = rows[slot, e, pl.ds(r * 16, 16)]
                plsc.addupdate(sumacc.at[dl, pl.ds(r * 16, 16)], rv)
                mv = maxacc[dl, pl.ds(r * 16, 16)]
                maxacc[dl, pl.ds(r * 16, 16)] = jnp.maximum(mv, rv)
                nv = minacc[dl, pl.ds(r * 16, 16)]
                minacc[dl, pl.ds(r * 16, 16)] = jnp.minimum(nv, rv)
            if with_cnt:
                plsc.addupdate(cntacc.at[dl, pl.ds(0, 16)], onev)
        return 0
    lax.fori_loop(0, CH // 16, _egrp, 0)


def _stream_pass(wid, feat_h, lpack, ncht, pbig, sbig, dbig, rows,
                 sumacc, maxacc, minacc, cntacc, with_cnt, onev, sems):
    nsb = (ncht + SB - 1) // SB

    def _gather(k, slot, sem):
        off = pl.multiple_of(k * CH, CH)
        pltpu.async_copy(feat_h.at[sbig.at[pl.ds(off, CH)]],
                         rows.at[slot], sem)

    def _gwait(slot, sem):
        pltpu.make_async_copy(feat_h.at[sbig.at[pl.ds(0, CH)]],
                              rows.at[slot], sem).wait()

    def _sb(sb, _):
        base = pl.multiple_of(sb * SBCH, SBCH)
        pltpu.sync_copy(lpack.at[wid, pl.ds(base, SBCH)], pbig)

        def _unp(g, _):
            pk = pbig[pl.ds(g * 16, 16)]
            sbig[pl.ds(g * 16, 16)] = lax.shift_right_logical(pk, 9)
            dbig[pl.ds(g * 16, 16)] = lax.bitwise_and(pk, 511)
            return 0
        lax.fori_loop(0, SBCH // 16, _unp, 0)

        csb = jnp.minimum(ncht - sb * SB, SB)
        for d in range(D):
            @pl.when(d < csb)
            def _():
                _gather(d, d, sems[d])

        def _grp(t, _):
            for d in range(D):
                k = t * D + d

                @pl.when(k < csb)
                def _():
                    _gwait(d, sems[d])
                    _accum_chunk(rows, d, dbig, k * CH, sumacc, maxacc,
                                 minacc, cntacc, with_cnt, onev)

                    @pl.when(k + D < csb)
                    def _():
                        _gather(k + D, d, sems[d])
            return 0
        lax.fori_loop(0, (SB + D - 1) // D, _grp, 0)
        return 0
    lax.fori_loop(0, nsb, _sb, 0)


def _sc_body(has_lists, *refs):
    if has_lists:
        (fa_hbm, fb_hbm, lpack_hbm, lcnt,
         sum_a, sum_b, mx_a, mx_b, mn_a, mn_b,
         src2, dst2, plist, pbig, sbig, dbig, rows,
         sumacc, maxacc, minacc, cntacc, cbuf,
         semb, semm, semg0, semg1, semg2) = refs
        cnt_o = None
    else:
        (fa_hbm, fb_hbm, src_hbm, dst_hbm,
         sum_a, sum_b, mx_a, mx_b, mn_a, mn_b, cnt_o, lpack_hbm, lcnt,
         src2, dst2, plist, pbig, sbig, dbig, rows,
         sumacc, maxacc, minacc, cntacc, cbuf,
         semb, semm, semg0, semg1, semg2) = refs
    sems = (semg0, semg1, semg2)

    c = lax.axis_index("c")
    s = lax.axis_index("s")
    wid = s * NC + c
    lo = wid * NPT
    hi = jnp.minimum(lo + NPT, N)

    negv = jnp.full((16,), NEG, f32)
    posv = jnp.full((16,), POS, f32)
    zerv = jnp.zeros((16,), f32)
    onev = jnp.ones((16,), f32)
    sentp = jnp.full((16,), TRASH, i32)

    if not has_lists:
        def _bfetch(b, slot):
            pltpu.async_copy(src_hbm.at[pl.ds(b * KE, KE)],
                             src2.at[slot], semb)
            pltpu.async_copy(dst_hbm.at[pl.ds(b * KE, KE)],
                             dst2.at[slot], semb)

        _bfetch(0, 0)

        def _block(b, mtot):
            for _ in range(2):
                pltpu.make_async_copy(src_hbm.at[pl.ds(0, KE)],
                                      src2.at[0], semb).wait()
            slot = lax.rem(b, BR)

            def _filt(i, mc):
                dv = dst2[slot, pl.ds(i * 16, 16)]
                sv = src2[slot, pl.ds(i * 16, 16)]
                m = (dv >= lo) & (dv < hi)
                dl = jnp.minimum(dv - lo, TRASH)
                pk = lax.bitwise_or(lax.shift_left(sv, 9), dl)
                plsc.store_compressed(plist.at[pl.ds(mc, 16)], pk, mask=m)
                return mc + plsc.all_reduce_population_count(m)[0]
            mc = lax.fori_loop(0, KE // 16, _filt, jnp.int32(0))

            for p in range(CH // 16):
                plist[pl.ds(mc + p * 16, 16)] = sentp

            nch = (mc + CH - 1) // CH

            def _mirror(j, _):
                off = pl.multiple_of(mtot + j * CH, CH)
                src_off = pl.multiple_of(j * CH, CH)
                pltpu.async_copy(plist.at[pl.ds(src_off, CH)],
                                 lpack_hbm.at[wid, pl.ds(off, CH)], semm)
                return 0
            lax.fori_loop(0, nch, _mirror, 0)

            @pl.when(b + 1 < NBLK)
            def _():
                _bfetch(b + 1, lax.rem(b + 1, BR))
            return mtot + nch * CH
        mtot = lax.fori_loop(0, NBLK, _block, jnp.int32(0))
        ncht = mtot // CH
        cbuf[pl.ds(0, 16)] = jnp.full((16,), ncht, i32)
        pltpu.sync_copy(cbuf, lcnt.at[wid])

        def _drain(j, _):
            pltpu.make_async_copy(plist.at[pl.ds(0, CH)],
                                  lpack_hbm.at[0, pl.ds(0, CH)],
                                  semm).wait()
            return 0
        lax.fori_loop(0, ncht, _drain, 0)
    else:
        pltpu.sync_copy(lcnt.at[wid], cbuf)
        ncht = cbuf[pl.ds(0, 16)][0]

    for half, (feat_h, sum_h, mx_h, mn_h) in enumerate(
            ((fa_hbm, sum_a, mx_a, mn_a), (fb_hbm, sum_b, mx_b, mn_b))):
        with_cnt = (not has_lists) and half == 0

        def _init_row(i, _):
            for r in range(FH // 16):
                sumacc[i, pl.ds(r * 16, 16)] = zerv
                maxacc[i, pl.ds(r * 16, 16)] = negv
                minacc[i, pl.ds(r * 16, 16)] = posv
            if with_cnt:
                cntacc[i, pl.ds(0, 16)] = zerv
            return 0
        lax.fori_loop(0, ACC_ROWS, _init_row, 0)

        _stream_pass(wid, feat_h, lpack_hbm, ncht, pbig, sbig, dbig, rows,
                     sumacc, maxacc, minacc, cntacc, with_cnt, onev, sems)

        pltpu.sync_copy(sumacc.at[pl.ds(0, NPT)], sum_h.at[pl.ds(lo, NPT)])
        pltpu.sync_copy(maxacc.at[pl.ds(0, NPT)], mx_h.at[pl.ds(lo, NPT)])
        pltpu.sync_copy(minacc.at[pl.ds(0, NPT)], mn_h.at[pl.ds(lo, NPT)])
        if with_cnt:
            pltpu.sync_copy(cntacc.at[pl.ds(0, NPT)],
                            cnt_o.at[pl.ds(lo, NPT)])


_AGG_OUTS = [jax.ShapeDtypeStruct((NW * NPT, FH), f32)] * 6

_SCRATCH = [
    pltpu.VMEM((BR, KE), i32),
    pltpu.VMEM((BR, KE), i32),
    pltpu.VMEM((KE + CH,), i32),
    pltpu.VMEM((SBCH,), i32),
    pltpu.VMEM((SBCH,), i32),
    pltpu.VMEM((SBCH,), i32),
    pltpu.VMEM((D, CH, FH), f32),
    pltpu.VMEM((ACC_ROWS, FH), f32),
    pltpu.VMEM((ACC_ROWS, FH), f32),
    pltpu.VMEM((ACC_ROWS, FH), f32),
    pltpu.VMEM((ACC_ROWS, 16), f32),
    pltpu.VMEM((16,), i32),
    pltpu.SemaphoreType.DMA,
    pltpu.SemaphoreType.DMA,
    pltpu.SemaphoreType.DMA,
    pltpu.SemaphoreType.DMA,
    pltpu.SemaphoreType.DMA,
]

_MESH_KW = dict(
    mesh=plsc.VectorSubcoreMesh(core_axis_name="c", subcore_axis_name="s"),
    compiler_params=pltpu.CompilerParams(needs_layout_passes=False,
                                         use_tc_tiling_on_sc=False),
)

_sc_agg_build = functools.partial(
    pl.kernel,
    out_type=_AGG_OUTS + [
        jax.ShapeDtypeStruct((NW * NPT, 16), f32),
        jax.ShapeDtypeStruct((NW, EPAD), i32),
        jax.ShapeDtypeStruct((NW, 16), i32),
    ],
    scratch_types=_SCRATCH,
    **_MESH_KW,
)(functools.partial(_sc_body, False))

_sc_agg_reuse = functools.partial(
    pl.kernel,
    out_type=list(_AGG_OUTS),
    scratch_types=_SCRATCH,
    **_MESH_KW,
)(functools.partial(_sc_body, True))


BLK = 1000
NGRID = N // BLK


def _sage_block(refs, cnt):
    (sa_ref, sb_ref, xa_ref, xb_ref, na_ref, nb_ref, x_ref,
     wma_ref, wmb_ref, wxa_ref, wxb_ref, wna_ref, wnb_ref,
     wr_ref, b_ref) = refs
    rinv = 1.0 / jnp.maximum(cnt, 1.0)
    has = cnt > 0.0
    dot = functools.partial(jnp.dot, preferred_element_type=f32)
    h = (dot(sa_ref[...] * rinv, wma_ref[...])
         + dot(sb_ref[...] * rinv, wmb_ref[...])
         + dot(jnp.where(has, xa_ref[...], 0.0), wxa_ref[...])
         + dot(jnp.where(has, xb_ref[...], 0.0), wxb_ref[...])
         + dot(jnp.where(has, na_ref[...], 0.0), wna_ref[...])
         + dot(jnp.where(has, nb_ref[...], 0.0), wnb_ref[...])
         + dot(x_ref[...], wr_ref[...])
         + b_ref[...])
    return jnp.maximum(h, 0.0)


def _tc_layer_body(sa_ref, sb_ref, xa_ref, xb_ref, na_ref, nb_ref,
                   cnt_ref, x_ref,
                   wma_ref, wmb_ref, wxa_ref, wxb_ref, wna_ref, wnb_ref,
                   wr_ref, b_ref, oa_ref, ob_ref):
    cnt = cnt_ref[:, 0:1]
    h = _sage_block(
        (sa_ref, sb_ref, xa_ref, xb_ref, na_ref, nb_ref, x_ref,
         wma_ref, wmb_ref, wxa_ref, wxb_ref, wna_ref, wnb_ref,
         wr_ref, b_ref), cnt)
    oa_ref[...] = h[:, :FH]
    ob_ref[...] = h[:, FH:]


def _tc_layer(sa, sb, xa, xb, na, nb, cnt, x, wma, wmb, wxa, wxb,
              wna, wnb, wr, b):
    half = pl.BlockSpec((BLK, FH), lambda i: (i, 0))
    wspec = pl.BlockSpec((FH, F), lambda i: (0, 0))
    return pl.pallas_call(
        _tc_layer_body,
        grid=(NGRID,),
        in_specs=[
            half, half, half, half, half, half,
            pl.BlockSpec((BLK, 16), lambda i: (i, 0)),
            pl.BlockSpec((BLK, F), lambda i: (i, 0)),
            wspec, wspec, wspec, wspec, wspec, wspec,
            pl.BlockSpec((F, F), lambda i: (0, 0)),
            pl.BlockSpec((1, F), lambda i: (0, 0)),
        ],
        out_specs=[half, half],
        out_shape=[jax.ShapeDtypeStruct((N, FH), f32),
                   jax.ShapeDtypeStruct((N, FH), f32)],
    )(sa, sb, xa, xb, na, nb, cnt, x, wma, wmb, wxa, wxb, wna, wnb, wr, b)


def _tc_final_body(sa_ref, sb_ref, xa_ref, xb_ref, na_ref, nb_ref,
                   cnt_ref, ha_ref, hb_ref,
                   wma_ref, wmb_ref, wxa_ref, wxb_ref, wna_ref, wnb_ref,
                   wra_ref, wrb_ref, b_ref,
                   batch_ref, wl_ref, bl_ref, wo_ref, bo_ref,
                   o_ref, pool_ref, cntb_ref):
    i = pl.program_id(0)

    @pl.when(i == 0)
    def _():
        pool_ref[...] = jnp.full((NB, F), NEG, f32)
        cntb_ref[...] = jnp.zeros((NB, 1), f32)

    cnt = cnt_ref[:, 0:1]
    rinv = 1.0 / jnp.maximum(cnt, 1.0)
    has = cnt > 0.0
    dot = functools.partial(jnp.dot, preferred_element_type=f32)
    h = (dot(sa_ref[...] * rinv, wma_ref[...])
         + dot(sb_ref[...] * rinv, wmb_ref[...])
         + dot(jnp.where(has, xa_ref[...], 0.0), wxa_ref[...])
         + dot(jnp.where(has, xb_ref[...], 0.0), wxb_ref[...])
         + dot(jnp.where(has, na_ref[...], 0.0), wna_ref[...])
         + dot(jnp.where(has, nb_ref[...], 0.0), wnb_ref[...])
         + dot(ha_ref[...], wra_ref[...])
         + dot(hb_ref[...], wrb_ref[...])
         + b_ref[...])
    h = jnp.maximum(h, 0.0)

    bids = batch_ref[...]
    iota = lax.broadcasted_iota(i32, (1, NB), 1).astype(f32)
    oh = bids == iota
    for b in range(NB):
        mb = oh[:, b:b + 1]
        contrib = jnp.max(jnp.where(mb, h, NEG), axis=0, keepdims=True)
        pool_ref[b:b + 1, :] = jnp.maximum(pool_ref[b:b + 1, :], contrib)
        cb = jnp.sum(mb.astype(f32))
        cntb_ref[b:b + 1, :] = cntb_ref[b:b + 1, :] + cb

    @pl.when(i == NGRID - 1)
    def _():
        pooled = jnp.where(cntb_ref[...] > 0.0, pool_ref[...], 0.0)
        z = dot(pooled, wl_ref[...]) + bl_ref[...]
        o_ref[...] = dot(z, wo_ref[...]) + bo_ref[...]


def _tc_final(sa, sb, xa, xb, na, nb, cnt, ha, hb,
              wma, wmb, wxa, wxb, wna, wnb, wra, wrb, b,
              batchf, wl, bl, wo, bo, dlin, dout):
    half = pl.BlockSpec((BLK, FH), lambda i: (i, 0))
    wspec = pl.BlockSpec((FH, F), lambda i: (0, 0))
    return pl.pallas_call(
        _tc_final_body,
        grid=(NGRID,),
        in_specs=[
            half, half, half, half, half, half,
            pl.BlockSpec((BLK, 16), lambda i: (i, 0)),
            half, half,
            wspec, wspec, wspec, wspec, wspec, wspec, wspec, wspec,
            pl.BlockSpec((1, F), lambda i: (0, 0)),
            pl.BlockSpec((BLK, 1), lambda i: (i, 0)),
            pl.BlockSpec((F, dlin), lambda i: (0, 0)),
            pl.BlockSpec((1, dlin), lambda i: (0, 0)),
            pl.BlockSpec((dlin, dout), lambda i: (0, 0)),
            pl.BlockSpec((1, dout), lambda i: (0, 0)),
        ],
        out_specs=pl.BlockSpec((NB, dout), lambda i: (0, 0)),
        out_shape=jax.ShapeDtypeStruct((NB, dout), f32),
        scratch_shapes=[
            pltpu.VMEM((NB, F), f32),
            pltpu.VMEM((NB, 1), f32),
        ],
    )(sa, sb, xa, xb, na, nb, cnt, ha, hb,
      wma, wmb, wxa, wxb, wna, wnb, wra, wrb, b,
      batchf, wl, bl, wo, bo)


def kernel(x, edge_index, batch, W_agg0, b_agg0, W_root0,
           W_agg1, b_agg1, W_root1, W_lin, b_lin, W_out, b_out):
    src = edge_index[0]
    dst = edge_index[1]
    xa = x[:, :FH]
    xb = x[:, FH:]

    (sa0, sb0, xma0, xmb0, mna0, mnb0, cnt,
     lpack, lcnt) = _sc_agg_build(xa, xb, src, dst)
    h1a, h1b = _tc_layer(
        sa0, sb0, xma0, xmb0, mna0, mnb0, cnt, x,
        W_agg0[:FH], W_agg0[FH:F], W_agg0[F:F + FH], W_agg0[F + FH:2 * F],
        W_agg0[2 * F:2 * F + FH], W_agg0[2 * F + FH:], W_root0,
        b_agg0.reshape(1, F))

    sa1, sb1, xma1, xmb1, mna1, mnb1 = _sc_agg_reuse(
        h1a, h1b, lpack, lcnt)
    batchf = batch.astype(f32).reshape(N, 1)
    dlin = W_lin.shape[1]
    dout = W_out.shape[1]
    out = _tc_final(
        sa1, sb1, xma1, xmb1, mna1, mnb1, cnt, h1a, h1b,
        W_agg1[:FH], W_agg1[FH:F], W_agg1[F:F + FH], W_agg1[F + FH:2 * F],
        W_agg1[2 * F:2 * F + FH], W_agg1[2 * F + FH:],
        W_root1[:FH], W_root1[FH:], b_agg1.reshape(1, F),
        batchf, W_lin, b_lin.reshape(1, dlin),
        W_out, b_out.reshape(1, dout), dlin, dout)
    return out


# speedup vs baseline: 7.0245x; 7.0245x over previous
import functools

import jax
import jax.numpy as jnp
from jax import lax
from jax.experimental import pallas as pl
from jax.experimental.pallas import tpu as pltpu
from jax.experimental.pallas import tpu_sc as plsc

N = 10000
E = 320000
F = 128
FH = 64
NB = 16

NC = 2
NS = 16
NW = NC * NS

NPT = 320
TRASH = NPT
ACC_ROWS = NPT + 1
KE = 4000
NBLK = E // KE
CH = 128
CPB = (KE + CH - 1) // CH
EPAD = NBLK * CPB * CH
SB = 16
SBCH = SB * CH
D = 2
SROWS = 10112
STRIPE = SROWS // NS

NEG = -3.0e38
POS = 3.0e38

f32 = jnp.float32
i32 = jnp.int32


def _tile_id():
    c = lax.axis_index("c")
    s = lax.axis_index("s")
    return s * NC + c, s


def _sc_filter_body(src_hbm, dst_hbm, lpack_hbm, lcnt,
                    src2, dst2, plist, cbuf, semb0, semb1, semm):
    wid, _ = _tile_id()
    lo = wid * NPT
    hi = jnp.minimum(lo + NPT, N)
    sentp = jnp.full((16,), TRASH, i32)

    def _fetch(b, slot, sem):
        pltpu.async_copy(src_hbm.at[pl.ds(b * KE, KE)], src2.at[slot], sem)
        pltpu.async_copy(dst_hbm.at[pl.ds(b * KE, KE)], dst2.at[slot], sem)

    _fetch(0, 0, semb0)
    _fetch(1, 1, semb1)

    def _mdrain(n):
        def _one(j, _):
            pltpu.make_async_copy(plist.at[0, pl.ds(0, CH)],
                                  lpack_hbm.at[0, pl.ds(0, CH)],
                                  semm).wait()
            return 0
        lax.fori_loop(0, n, _one, 0)

    def _group(gb, carry):
        mtot, nprev = carry
        nprev = list(nprev)
        for par in range(2):
            sem = (semb0, semb1)[par]
            b = gb * 2 + par
            for _ in range(2):
                pltpu.make_async_copy(src_hbm.at[pl.ds(0, KE)],
                                      src2.at[par], sem).wait()
            _mdrain(nprev[par])

            def _filt(i, mc):
                dv = dst2[par, pl.ds(i * 16, 16)]
                sv = src2[par, pl.ds(i * 16, 16)]
                m = (dv >= lo) & (dv < hi)
                dl = jnp.minimum(dv - lo, TRASH)
                pk = lax.bitwise_or(lax.shift_left(sv, 9), dl)
                plsc.store_compressed(plist.at[par, pl.ds(mc, 16)], pk,
                                      mask=m)
                return mc + plsc.all_reduce_population_count(m)[0]
            mc = lax.fori_loop(0, KE // 16, _filt, jnp.int32(0))

            for p in range(CH // 16):
                plist[par, pl.ds(mc + p * 16, 16)] = sentp

            nch = (mc + CH - 1) // CH

            def _mirror(j, _):
                off = pl.multiple_of(mtot + j * CH, CH)
                soff = pl.multiple_of(j * CH, CH)
                pltpu.async_copy(plist.at[par, pl.ds(soff, CH)],
                                 lpack_hbm.at[wid, pl.ds(off, CH)], semm)
                return 0
            lax.fori_loop(0, nch, _mirror, 0)
            mtot = mtot + nch * CH
            nprev[par] = nch

            @pl.when(b + 2 < NBLK)
            def _():
                _fetch(b + 2, par, sem)
        return mtot, tuple(nprev)
    mtot, nprev = lax.fori_loop(
        0, NBLK // 2, _group, (jnp.int32(0), (jnp.int32(0), jnp.int32(0))))
    ncht = mtot // CH
    cbuf[pl.ds(0, 16)] = jnp.full((16,), ncht, i32)
    pltpu.sync_copy(cbuf, lcnt.at[wid])
    _mdrain(nprev[0] + nprev[1])


def _accum_chunk(rows, slot, dbuf, doff, sumacc, maxacc, minacc, cntacc,
                 with_cnt, onev):
    def _egrp(g, _):
        dl16 = dbuf[pl.ds(doff + g * 16, 16)]
        for lane in range(16):
            dl = dl16[lane]
            e = g * 16 + lane
            for r in range(FH // 16):
                rv = rows[slot, e, pl.ds(r * 16, 16)]
                plsc.addupdate(sumacc.at[dl, pl.ds(r * 16, 16)], rv)
                mv = maxacc[dl, pl.ds(r * 16, 16)]
                maxacc[dl, pl.ds(r * 16, 16)] = jnp.maximum(mv, rv)
                nv = minacc[dl, pl.ds(r * 16, 16)]
                minacc[dl, pl.ds(r * 16, 16)] = jnp.minimum(nv, rv)
            if with_cnt:
                plsc.addupdate(cntacc.at[dl, pl.ds(0, 16)], onev)
        return 0
    lax.fori_loop(0, CH // 16, _egrp, 0)


def _stream_pass(wid, shx, lpack, ncht, pbig, sbig, dbig, rows,
                 sumacc, maxacc, minacc, cntacc, with_cnt, onev, sems):
    nsb = (ncht + SB - 1) // SB

    def _gather(k, slot, sem):
        off = pl.multiple_of(k * CH, CH)
        pltpu.async_copy(shx.at[sbig.at[pl.ds(off, CH)]],
                         rows.at[slot], sem)

    def _gwait(slot, sem):
        pltpu.make_async_copy(shx.at[sbig.at[pl.ds(0, CH)]],
                              rows.at[slot], sem).wait()

    def _sb(sb, _):
        base = pl.multiple_of(sb * SBCH, SBCH)
        pltpu.sync_copy(lpack.at[wid, pl.ds(base, SBCH)], pbig)

        def _unp(g, _):
            pk = pbig[pl.ds(g * 16, 16)]
            sbig[pl.ds(g * 16, 16)] = lax.shift_right_logical(pk, 9)
            dbig[pl.ds(g * 16, 16)] = lax.bitwise_and(pk, 511)
            return 0
        lax.fori_loop(0, SBCH // 16, _unp, 0)

        csb = jnp.minimum(ncht - sb * SB, SB)
        for d in range(D):
            @pl.when(d < csb)
            def _():
                _gather(d, d, sems[d])

        def _grp(t, _):
            for d in range(D):
                k = t * D + d

                @pl.when(k < csb)
                def _():
                    _gwait(d, sems[d])
                    _accum_chunk(rows, d, dbig, k * CH, sumacc, maxacc,
                                 minacc, cntacc, with_cnt, onev)

                    @pl.when(k + D < csb)
                    def _():
                        _gather(k + D, d, sems[d])
            return 0
        lax.fori_loop(0, (SB + D - 1) // D, _grp, 0)
        return 0
    lax.fori_loop(0, nsb, _sb, 0)


def _sc_stream_body(with_cnt, *refs):
    if with_cnt:
        (fa_hbm, fb_hbm, lpack_hbm, lcnt,
         sum_a, sum_b, mx_a, mx_b, mn_a, mn_b, cnt_o,
         pbig, sbig, dbig, rows, sumacc, maxacc, minacc, cntacc, cbuf,
         shx, semg0, semg1) = refs
    else:
        (fa_hbm, fb_hbm, lpack_hbm, lcnt,
         sum_a, sum_b, mx_a, mx_b, mn_a, mn_b,
         pbig, sbig, dbig, rows, sumacc, maxacc, minacc, cntacc, cbuf,
         shx, semg0, semg1) = refs
        cnt_o = None
    sems = (semg0, semg1)

    wid, s = _tile_id()
    lo = wid * NPT

    negv = jnp.full((16,), NEG, f32)
    posv = jnp.full((16,), POS, f32)
    zerv = jnp.zeros((16,), f32)
    onev = jnp.ones((16,), f32)

    pltpu.sync_copy(lcnt.at[wid], cbuf)
    ncht = cbuf[pl.ds(0, 16)][0]

    for half, (feat_h, sum_h, mx_h, mn_h) in enumerate(
            ((fa_hbm, sum_a, mx_a, mn_a), (fb_hbm, sum_b, mx_b, mn_b))):
        half_cnt = with_cnt and half == 0

        pltpu.sync_copy(feat_h.at[pl.ds(s * STRIPE, STRIPE)],
                        shx.at[pl.ds(s * STRIPE, STRIPE)])

        def _init_row(i, _):
            for r in range(FH // 16):
                sumacc[i, pl.ds(r * 16, 16)] = zerv
                maxacc[i, pl.ds(r * 16, 16)] = negv
                minacc[i, pl.ds(r * 16, 16)] = posv
            if half_cnt:
                cntacc[i, pl.ds(0, 16)] = zerv
            return 0
        lax.fori_loop(0, ACC_ROWS, _init_row, 0)

        plsc.subcore_barrier()

        _stream_pass(wid, shx, lpack_hbm, ncht, pbig, sbig, dbig, rows,
                     sumacc, maxacc, minacc, cntacc, half_cnt, onev, sems)

        pltpu.sync_copy(sumacc.at[pl.ds(0, NPT)], sum_h.at[pl.ds(lo, NPT)])
        pltpu.sync_copy(maxacc.at[pl.ds(0, NPT)], mx_h.at[pl.ds(lo, NPT)])
        pltpu.sync_copy(minacc.at[pl.ds(0, NPT)], mn_h.at[pl.ds(lo, NPT)])
        if half_cnt:
            pltpu.sync_copy(cntacc.at[pl.ds(0, NPT)],
                            cnt_o.at[pl.ds(lo, NPT)])
        plsc.subcore_barrier()


_AGG_OUTS = [jax.ShapeDtypeStruct((NW * NPT, FH), f32)] * 6

_STREAM_SCRATCH = [
    pltpu.VMEM((SBCH,), i32),
    pltpu.VMEM((SBCH,), i32),
    pltpu.VMEM((SBCH,), i32),
    pltpu.VMEM((D, CH, FH), f32),
    pltpu.VMEM((ACC_ROWS, FH), f32),
    pltpu.VMEM((ACC_ROWS, FH), f32),
    pltpu.VMEM((ACC_ROWS, FH), f32),
    pltpu.VMEM((ACC_ROWS, 16), f32),
    pltpu.VMEM((16,), i32),
    pltpu.VMEM_SHARED((SROWS, FH), f32),
    pltpu.SemaphoreType.DMA,
    pltpu.SemaphoreType.DMA,
]

_MESH_KW = dict(
    mesh=plsc.VectorSubcoreMesh(core_axis_name="c", subcore_axis_name="s"),
    compiler_params=pltpu.CompilerParams(needs_layout_passes=False,
                                         use_tc_tiling_on_sc=False),
)

_sc_filter = functools.partial(
    pl.kernel,
    out_type=[
        jax.ShapeDtypeStruct((NW, EPAD), i32),
        jax.ShapeDtypeStruct((NW, 16), i32),
    ],
    scratch_types=[
        pltpu.VMEM((2, KE), i32),
        pltpu.VMEM((2, KE), i32),
        pltpu.VMEM((2, KE + CH), i32),
        pltpu.VMEM((16,), i32),
        pltpu.SemaphoreType.DMA,
        pltpu.SemaphoreType.DMA,
        pltpu.SemaphoreType.DMA,
    ],
    **_MESH_KW,
)(_sc_filter_body)

_sc_stream_cnt = functools.partial(
    pl.kernel,
    out_type=_AGG_OUTS + [jax.ShapeDtypeStruct((NW * NPT, 16), f32)],
    scratch_types=list(_STREAM_SCRATCH),
    **_MESH_KW,
)(functools.partial(_sc_stream_body, True))

_sc_stream = functools.partial(
    pl.kernel,
    out_type=list(_AGG_OUTS),
    scratch_types=list(_STREAM_SCRATCH),
    **_MESH_KW,
)(functools.partial(_sc_stream_body, False))


BLK = 1000
NGRID = N // BLK


def _tc_layer_body(sa_ref, sb_ref, xa_ref, xb_ref, na_ref, nb_ref,
                   cnt_ref, x_ref,
                   wma_ref, wmb_ref, wxa_ref, wxb_ref, wna_ref, wnb_ref,
                   wr_ref, b_ref, oa_ref, ob_ref):
    cnt = cnt_ref[:, 0:1]
    rinv = 1.0 / jnp.maximum(cnt, 1.0)
    has = cnt > 0.0
    dot = functools.partial(jnp.dot, preferred_element_type=f32)
    h = (dot(sa_ref[...] * rinv, wma_ref[...])
         + dot(sb_ref[...] * rinv, wmb_ref[...])
         + dot(jnp.where(has, xa_ref[...], 0.0), wxa_ref[...])
         + dot(jnp.where(has, xb_ref[...], 0.0), wxb_ref[...])
         + dot(jnp.where(has, na_ref[...], 0.0), wna_ref[...])
         + dot(jnp.where(has, nb_ref[...], 0.0), wnb_ref[...])
         + dot(x_ref[...], wr_ref[...])
         + b_ref[...])
    h = jnp.maximum(h, 0.0)
    oa_ref[...] = h[:, :FH]
    ob_ref[...] = h[:, FH:]


def _tc_layer(sa, sb, xa, xb, na, nb, cnt, x, wma, wmb, wxa, wxb,
              wna, wnb, wr, b):
    half = pl.BlockSpec((BLK, FH), lambda i: (i, 0))
    wspec = pl.BlockSpec((FH, F), lambda i: (0, 0))
    return pl.pallas_call(
        _tc_layer_body,
        grid=(NGRID,),
        in_specs=[
            half, half, half, half, half, half,
            pl.BlockSpec((BLK, 16), lambda i: (i, 0)),
            pl.BlockSpec((BLK, F), lambda i: (i, 0)),
            wspec, wspec, wspec, wspec, wspec, wspec,
            pl.BlockSpec((F, F), lambda i: (0, 0)),
            pl.BlockSpec((1, F), lambda i: (0, 0)),
        ],
        out_specs=[half, half],
        out_shape=[jax.ShapeDtypeStruct((SROWS, FH), f32),
                   jax.ShapeDtypeStruct((SROWS, FH), f32)],
    )(sa, sb, xa, xb, na, nb, cnt, x, wma, wmb, wxa, wxb, wna, wnb, wr, b)


def _tc_final_body(sa_ref, sb_ref, xa_ref, xb_ref, na_ref, nb_ref,
                   cnt_ref, ha_ref, hb_ref,
                   wma_ref, wmb_ref, wxa_ref, wxb_ref, wna_ref, wnb_ref,
                   wra_ref, wrb_ref, b_ref,
                   batch_ref, wl_ref, bl_ref, wo_ref, bo_ref,
                   o_ref, pool_ref, cntb_ref):
    i = pl.program_id(0)

    @pl.when(i == 0)
    def _():
        pool_ref[...] = jnp.full((NB, F), NEG, f32)
        cntb_ref[...] = jnp.zeros((NB, 1), f32)

    cnt = cnt_ref[:, 0:1]
    rinv = 1.0 / jnp.maximum(cnt, 1.0)
    has = cnt > 0.0
    dot = functools.partial(jnp.dot, preferred_element_type=f32)
    h = (dot(sa_ref[...] * rinv, wma_ref[...])
         + dot(sb_ref[...] * rinv, wmb_ref[...])
         + dot(jnp.where(has, xa_ref[...], 0.0), wxa_ref[...])
         + dot(jnp.where(has, xb_ref[...], 0.0), wxb_ref[...])
         + dot(jnp.where(has, na_ref[...], 0.0), wna_ref[...])
         + dot(jnp.where(has, nb_ref[...], 0.0), wnb_ref[...])
         + dot(ha_ref[...], wra_ref[...])
         + dot(hb_ref[...], wrb_ref[...])
         + b_ref[...])
    h = jnp.maximum(h, 0.0)

    bids = batch_ref[...]
    iota = lax.broadcasted_iota(i32, (1, NB), 1).astype(f32)
    oh = bids == iota
    for b in range(NB):
        mb = oh[:, b:b + 1]
        contrib = jnp.max(jnp.where(mb, h, NEG), axis=0, keepdims=True)
        pool_ref[b:b + 1, :] = jnp.maximum(pool_ref[b:b + 1, :], contrib)
        cb = jnp.sum(mb.astype(f32))
        cntb_ref[b:b + 1, :] = cntb_ref[b:b + 1, :] + cb

    @pl.when(i == NGRID - 1)
    def _():
        pooled = jnp.where(cntb_ref[...] > 0.0, pool_ref[...], 0.0)
        z = dot(pooled, wl_ref[...]) + bl_ref[...]
        o_ref[...] = dot(z, wo_ref[...]) + bo_ref[...]


def _tc_final(sa, sb, xa, xb, na, nb, cnt, ha, hb,
              wma, wmb, wxa, wxb, wna, wnb, wra, wrb, b,
              batchf, wl, bl, wo, bo, dlin, dout):
    half = pl.BlockSpec((BLK, FH), lambda i: (i, 0))
    wspec = pl.BlockSpec((FH, F), lambda i: (0, 0))
    return pl.pallas_call(
        _tc_final_body,
        grid=(NGRID,),
        in_specs=[
            half, half, half, half, half, half,
            pl.BlockSpec((BLK, 16), lambda i: (i, 0)),
            half, half,
            wspec, wspec, wspec, wspec, wspec, wspec, wspec, wspec,
            pl.BlockSpec((1, F), lambda i: (0, 0)),
            pl.BlockSpec((BLK, 1), lambda i: (i, 0)),
            pl.BlockSpec((F, dlin), lambda i: (0, 0)),
            pl.BlockSpec((1, dlin), lambda i: (0, 0)),
            pl.BlockSpec((dlin, dout), lambda i: (0, 0)),
            pl.BlockSpec((1, dout), lambda i: (0, 0)),
        ],
        out_specs=pl.BlockSpec((NB, dout), lambda i: (0, 0)),
        out_shape=jax.ShapeDtypeStruct((NB, dout), f32),
        scratch_shapes=[
            pltpu.VMEM((NB, F), f32),
            pltpu.VMEM((NB, 1), f32),
        ],
    )(sa, sb, xa, xb, na, nb, cnt, ha, hb,
      wma, wmb, wxa, wxb, wna, wnb, wra, wrb, b,
      batchf, wl, bl, wo, bo)


def kernel(x, edge_index, batch, W_agg0, b_agg0, W_root0,
           W_agg1, b_agg1, W_root1, W_lin, b_lin, W_out, b_out):
    src = edge_index[0]
    dst = edge_index[1]
    xa = jnp.pad(x[:, :FH], ((0, SROWS - N), (0, 0)))
    xb = jnp.pad(x[:, FH:], ((0, SROWS - N), (0, 0)))

    lpack, lcnt = _sc_filter(src, dst)
    (sa0, sb0, xma0, xmb0, mna0, mnb0,
     cnt) = _sc_stream_cnt(xa, xb, lpack, lcnt)
    h1a, h1b = _tc_layer(
        sa0, sb0, xma0, xmb0, mna0, mnb0, cnt, x,
        W_agg0[:FH], W_agg0[FH:F], W_agg0[F:F + FH], W_agg0[F + FH:2 * F],
        W_agg0[2 * F:2 * F + FH], W_agg0[2 * F + FH:], W_root0,
        b_agg0.reshape(1, F))

    sa1, sb1, xma1, xmb1, mna1, mnb1 = _sc_stream(h1a, h1b, lpack, lcnt)
    batchf = batch.astype(f32).reshape(N, 1)
    dlin = W_lin.shape[1]
    dout = W_out.shape[1]
    out = _tc_final(
        sa1, sb1, xma1, xmb1, mna1, mnb1, cnt, h1a[:N], h1b[:N],
        W_agg1[:FH], W_agg1[FH:F], W_agg1[F:F + FH], W_agg1[F + FH:2 * F],
        W_agg1[2 * F:2 * F + FH], W_agg1[2 * F + FH:],
        W_root1[:FH], W_root1[FH:], b_agg1.reshape(1, F),
        batchf, W_lin, b_lin.reshape(1, dlin),
        W_out, b_out.reshape(1, dout), dlin, dout)
    return out
